# baseline (speedup 1.0000x reference)
import jax
import jax.numpy as jnp
from jax import lax
from jax.experimental import pallas as pl
from jax.experimental.pallas import tpu as pltpu
from jax.experimental.pallas import tpu_sc as plsc

N_GRID = 64
N1 = N_GRID + 1
V = N1 * N1 * N1
D = 128
B = 262144
L = 16

NC = 2
NS = 16
NW = NC * NS
PT = B // NW
C = 64
NCHUNK = PT // C

_OFFS = tuple(ox * (N1 * N1) + oy * N1 + oz
              for ox in (0, 1) for oy in (0, 1) for oz in (0, 1))
_PAIR_OFFS = tuple(ox * (N1 * N1) + oy * N1 for ox in (0, 1) for oy in (0, 1))


def _body(xT, valt, ptab, outv_hbm, outf_hbm, xv, *bufs_flat):
    semg = bufs_flat[-4:-2]
    semo = bufs_flat[-2:]
    bufs = (bufs_flat[0:8], bufs_flat[8:16])

    wid = lax.axis_index("s") * NC + lax.axis_index("c")
    base = wid * PT

    for d in range(3):
        pltpu.sync_copy(xT.at[pl.ds(d * B + base, PT)],
                        xv.at[pl.ds(d * PT, PT)])

    def compute_idx(i, idxb, vidxb, wb, wpb):
        off = i * C
        for g in range(C // L):
            s = off + g * L
            px = (xv[pl.ds(s, L)] + 1.0) * 32.0
            py = (xv[pl.ds(PT + s, L)] + 1.0) * 32.0
            pz = (xv[pl.ds(2 * PT + s, L)] + 1.0) * 32.0
            ix = px.astype(jnp.int32)
            iy = py.astype(jnp.int32)
            iz = pz.astype(jnp.int32)
            fx = px - ix.astype(jnp.float32)
            fy = py - iy.astype(jnp.float32)
            fz = pz - iz.astype(jnp.float32)
            b0 = ix * (N1 * N1) + iy * N1 + iz
            for c2 in range(4):
                idxb[c2, pl.ds(g * L, L)] = b0 + _PAIR_OFFS[c2]
            cidx = 0
            for ox in (0, 1):
                wx = fx if ox else 1.0 - fx
                for oy in (0, 1):
                    wxy = wx * (fy if oy else 1.0 - fy)
                    for oz in (0, 1):
                        w = wxy * (fz if oz else 1.0 - fz)
                        vidxb[cidx, pl.ds(g * L, L)] = b0 + _OFFS[cidx]
                        wb[pl.ds(cidx * C + g * L, L)] = w
                        wp = plsc.pack(w, w,
                                       format=plsc.PackFormat.INTERLEAVED)
                        wpb[pl.ds(cidx * C + g * L, L)] = plsc.bitcast(
                            wp, jnp.int32)
                        cidx += 1

    def fire_gathers(idxb, vidxb, rows, vrows, sem):
        for c2 in range(4):
            pltpu.async_copy(ptab.at[idxb.at[c2]],
                             rows.at[pl.ds(c2 * C, C)], sem)
        for c in range(8):
            pltpu.async_copy(valt.at[vidxb.at[c]],
                             vrows.at[pl.ds(c * C, C)], sem)

    def wait_gathers(rows, vrows, sem):
        pltpu.make_async_copy(ptab.at[pl.ds(0, 4 * C)], rows, sem).wait()
        pltpu.make_async_copy(valt.at[pl.ds(0, 8 * C)], vrows, sem).wait()

    def accumulate(wb, wpb, rows, vrows, outf, outv):
        for g in range(C // L):
            s = g * L
            acc = wb[pl.ds(s, L)] * vrows[pl.ds(s, L)]
            for c in range(1, 8):
                acc = acc + wb[pl.ds(c * C + s, L)] * vrows[pl.ds(c * C + s, L)]
            outv[pl.ds(s, L)] = acc

        ar2 = jnp.arange(0, 32, 2, dtype=jnp.int32)

        def pt(j, carry2):
            jv = jnp.full((L,), j, dtype=jnp.int32)
            acc = [None] * (D // 32)
            for c2 in range(4):
                r = c2 * C + j
                for oz in (0, 1):
                    ws = plsc.bitcast(
                        plsc.load_gather(wpb, [jv + ((2 * c2 + oz) * C)]),
                        jnp.bfloat16)
                    half = oz * (D // 2)
                    for k in range(D // 32):
                        rk = plsc.bitcast(rows[r, pl.ds(half + k * L, L)],
                                          jnp.bfloat16)
                        if c2 == 0 and oz == 0:
                            acc[k] = ws * rk
                        else:
                            acc[k] = acc[k] + ws * rk
            for k in range(D // 32):
                a, b = plsc.unpack(acc[k], format=plsc.PackFormat.INTERLEAVED)
                cola = ar2 + (32 * k)
                plsc.store_scatter(outf, [jv, cola], a)
                plsc.store_scatter(outf, [jv, cola + 1], b)
            return carry2

        lax.fori_loop(0, C, pt, 0, unroll=2)

    def fire_out(i, outf, outv, sem):
        t = base + i * C
        pltpu.async_copy(outf, outf_hbm.at[pl.ds(t, C)], sem)
        pltpu.async_copy(outv, outv_hbm.at[pl.ds(t, C)], sem)

    def wait_out(outf, outv, sem):
        pltpu.make_async_copy(outf, outf_hbm.at[pl.ds(base, C)], sem).wait()
        pltpu.make_async_copy(outv, outv_hbm.at[pl.ds(base, C)], sem).wait()

    idxb0, vidxb0, wb0, wpb0, rows0, vrows0, _, _ = bufs[0]
    compute_idx(0, idxb0, vidxb0, wb0, wpb0)
    fire_gathers(idxb0, vidxb0, rows0, vrows0, semg[0])

    def body2(k, carry):
        for p in (0, 1):
            i = 2 * k + p
            q = 1 - p
            idxb, vidxb, wb, wpb, rows, vrows, outf, outv = bufs[p]
            idxbq, vidxbq, wbq, wpbq, rowsq, vrowsq, _, _ = bufs[q]

            @pl.when(i + 1 < NCHUNK)
            def _prefetch():
                compute_idx(i + 1, idxbq, vidxbq, wbq, wpbq)
                fire_gathers(idxbq, vidxbq, rowsq, vrowsq, semg[q])

            wait_gathers(rows, vrows, semg[p])

            @pl.when(i >= 2)
            def _drain_out():
                wait_out(outf, outv, semo[p])

            accumulate(wb, wpb, rows, vrows, outf, outv)
            fire_out(i, outf, outv, semo[p])
        return carry

    lax.fori_loop(0, NCHUNK // 2, body2, 0)

    for p in (0, 1):
        outf, outv = bufs[p][6], bufs[p][7]
        wait_out(outf, outv, semo[p])


def _parity_bufs():
    return (
        pltpu.VMEM((4, C), jnp.int32),
        pltpu.VMEM((8, C), jnp.int32),
        pltpu.VMEM((8 * C,), jnp.float32),
        pltpu.VMEM((8 * C,), jnp.int32),
        pltpu.VMEM((4 * C, D), jnp.int32),
        pltpu.VMEM((8 * C,), jnp.float32),
        pltpu.VMEM((C, D), jnp.float32),
        pltpu.VMEM((C,), jnp.float32),
    )


_sc_call = pl.kernel(
    _body,
    out_type=(
        jax.ShapeDtypeStruct((B,), jnp.float32),
        jax.ShapeDtypeStruct((B, D), jnp.float32),
    ),
    mesh=plsc.VectorSubcoreMesh(core_axis_name="c", subcore_axis_name="s"),
    compiler_params=pltpu.CompilerParams(needs_layout_passes=False),
    scratch_types=(
        pltpu.VMEM((3 * PT,), jnp.float32),
        *_parity_bufs(),
        *_parity_bufs(),
        pltpu.SemaphoreType.DMA,
        pltpu.SemaphoreType.DMA,
        pltpu.SemaphoreType.DMA,
        pltpu.SemaphoreType.DMA,
    ),
)


@jax.jit
def kernel(x, grid_value_param, grid_feature_param):
    xT = x.T.reshape(-1)
    valt = grid_value_param.reshape(-1)
    u = lax.bitcast_convert_type(grid_feature_param, jnp.uint32)
    rb = (u + 0x7FFF + ((u >> 16) & 1)) >> 16
    packed = rb[:, 0::2] | (rb[:, 1::2] << 16)
    packed = lax.bitcast_convert_type(packed, jnp.int32)
    ptab = jnp.concatenate([packed[:-1], packed[1:]], axis=1)
    outv, outf = _sc_call(xT, valt, ptab)
    return outv.reshape(B, 1), outf


# speedup vs baseline: 8.7912x; 8.7912x over previous
import jax
import jax.numpy as jnp
from jax import lax
from jax.experimental import pallas as pl
from jax.experimental.pallas import tpu as pltpu
from jax.experimental.pallas import tpu_sc as plsc

N_GRID = 64
N1 = N_GRID + 1
V = N1 * N1 * N1
D = 128
B = 262144
L = 16

NC = 2
NS = 16
NW = NC * NS
PT = B // NW
C = 64
NCHUNK = PT // C

NR = V - 1
RC = 128
NFULL = 67
RT = NFULL * RC
TAIL0 = NW * RT
NTAIL = NR - TAIL0
TC2 = NTAIL // 2

_OFFS = tuple(ox * (N1 * N1) + oy * N1 + oz
              for ox in (0, 1) for oy in (0, 1) for oz in (0, 1))
_PAIR_OFFS = tuple(ox * (N1 * N1) + oy * N1 for ox in (0, 1) for oy in (0, 1))


def _pack_body(feat, tail8, ptab, fin0, fout0, fin1, fout1, semi0, semi1,
               semo0, semo1):
    wid = lax.axis_index("s") * NC + lax.axis_index("c")
    r0 = wid * RT
    fins = (fin0, fin1)
    fouts = (fout0, fout1)
    semi = (semi0, semi1)
    semo = (semo0, semo1)

    def fire_read(i, fin, sem):
        pltpu.async_copy(feat.at[pl.ds(r0 + i * RC, RC + 8)], fin, sem)

    def pack_chunk(fin, fout, n):
        def row(rr, carry):
            nxt = []
            for k in range(D // L):
                b = fin[rr + 1, pl.ds(k * L, L)]
                wp = plsc.pack(carry[k], b, format=plsc.PackFormat.INTERLEAVED)
                fout[rr, pl.ds(k * L, L)] = plsc.bitcast(wp, jnp.int32)
                nxt.append(b)
            return tuple(nxt)

        first = tuple(fin[0, pl.ds(k * L, L)] for k in range(D // L))
        lax.fori_loop(0, n, row, first, unroll=2)

    fire_read(0, fins[0], semi[0])

    def body2(kk, carry):
        for p in (0, 1):
            i = 2 * kk + p
            q = 1 - p

            @pl.when(i + 1 < NFULL)
            def _prefetch():
                fire_read(i + 1, fins[q], semi[q])

            pltpu.make_async_copy(feat.at[pl.ds(r0, RC + 8)], fins[p],
                                  semi[p]).wait()

            @pl.when(i >= 2)
            def _drain():
                pltpu.make_async_copy(
                    fouts[p], ptab.at[pl.ds(r0, RC)], semo[p]).wait()

            pack_chunk(fins[p], fouts[p], RC)
            pltpu.async_copy(fouts[p], ptab.at[pl.ds(r0 + i * RC, RC)],
                             semo[p])
        return carry

    lax.fori_loop(0, NFULL // 2, body2, 0)

    if NFULL % 2 == 1:
        i = NFULL - 1
        p = i % 2
        pltpu.make_async_copy(feat.at[pl.ds(r0, RC + 8)], fins[p],
                              semi[p]).wait()
        if i >= 2:
            pltpu.make_async_copy(fouts[p], ptab.at[pl.ds(r0, RC)],
                                  semo[p]).wait()
        pack_chunk(fins[p], fouts[p], RC)
        pltpu.async_copy(fouts[p], ptab.at[pl.ds(r0 + i * RC, RC)], semo[p])

    for p in (0, 1):
        pltpu.make_async_copy(fouts[p], ptab.at[pl.ds(r0, RC)],
                              semo[p]).wait()

    @pl.when(wid == NW - 1)
    def _tail():
        for tch in range(2):
            off = TAIL0 + tch * TC2
            if tch == 0:
                pltpu.sync_copy(feat.at[pl.ds(off, TC2 + 8)],
                                fins[0].at[pl.ds(0, TC2 + 8)])
            else:
                pltpu.sync_copy(feat.at[pl.ds(off, TC2)],
                                fins[0].at[pl.ds(0, TC2)])
                pltpu.sync_copy(tail8, fins[0].at[pl.ds(TC2 - 7, 8)])
            pack_chunk(fins[0], fouts[0], TC2)
            pltpu.sync_copy(fouts[0].at[pl.ds(0, TC2)],
                            ptab.at[pl.ds(off, TC2)])


_pack_call = pl.kernel(
    _pack_body,
    out_type=jax.ShapeDtypeStruct((NR, D), jnp.int32),
    mesh=plsc.VectorSubcoreMesh(core_axis_name="c", subcore_axis_name="s"),
    compiler_params=pltpu.CompilerParams(needs_layout_passes=False),
    scratch_types=(
        pltpu.VMEM((RC + 8, D), jnp.float32),
        pltpu.VMEM((RC, D), jnp.int32),
        pltpu.VMEM((RC + 8, D), jnp.float32),
        pltpu.VMEM((RC, D), jnp.int32),
        pltpu.SemaphoreType.DMA,
        pltpu.SemaphoreType.DMA,
        pltpu.SemaphoreType.DMA,
        pltpu.SemaphoreType.DMA,
    ),
)


def _body(xT, valt, ptab, outv_hbm, outf_hbm, xv, *bufs_flat):
    semg = bufs_flat[-4:-2]
    semo = bufs_flat[-2:]
    bufs = (bufs_flat[0:8], bufs_flat[8:16])

    wid = lax.axis_index("s") * NC + lax.axis_index("c")
    base = wid * PT

    for d in range(3):
        pltpu.sync_copy(xT.at[pl.ds(d * B + base, PT)],
                        xv.at[pl.ds(d * PT, PT)])

    def compute_idx(i, idxb, vidxb, wb, wpb):
        off = i * C
        for g in range(C // L):
            s = off + g * L
            px = (xv[pl.ds(s, L)] + 1.0) * 32.0
            py = (xv[pl.ds(PT + s, L)] + 1.0) * 32.0
            pz = (xv[pl.ds(2 * PT + s, L)] + 1.0) * 32.0
            ix = px.astype(jnp.int32)
            iy = py.astype(jnp.int32)
            iz = pz.astype(jnp.int32)
            fx = px - ix.astype(jnp.float32)
            fy = py - iy.astype(jnp.float32)
            fz = pz - iz.astype(jnp.float32)
            b0 = ix * (N1 * N1) + iy * N1 + iz
            for c2 in range(4):
                idxb[c2, pl.ds(g * L, L)] = b0 + _PAIR_OFFS[c2]
            cidx = 0
            wz0 = 1.0 - fz
            for ox in (0, 1):
                wx = fx if ox else 1.0 - fx
                for oy in (0, 1):
                    wxy = wx * (fy if oy else 1.0 - fy)
                    wlo = wxy * wz0
                    whi = wxy * fz
                    vidxb[cidx, pl.ds(g * L, L)] = b0 + _OFFS[cidx]
                    wb[pl.ds(cidx * C + g * L, L)] = wlo
                    vidxb[cidx + 1, pl.ds(g * L, L)] = b0 + _OFFS[cidx + 1]
                    wb[pl.ds((cidx + 1) * C + g * L, L)] = whi
                    wp = plsc.pack(wlo, whi,
                                   format=plsc.PackFormat.INTERLEAVED)
                    wpb[pl.ds((cidx // 2) * C + g * L, L)] = plsc.bitcast(
                        wp, jnp.int32)
                    cidx += 2

    def fire_gathers(idxb, vidxb, rows, vrows, sem):
        for c2 in range(4):
            pltpu.async_copy(ptab.at[idxb.at[c2]],
                             rows.at[pl.ds(c2 * C, C)], sem)
        for c in range(8):
            pltpu.async_copy(valt.at[vidxb.at[c]],
                             vrows.at[pl.ds(c * C, C)], sem)

    def wait_gathers(rows, vrows, sem):
        pltpu.make_async_copy(ptab.at[pl.ds(0, 4 * C)], rows, sem).wait()
        pltpu.make_async_copy(valt.at[pl.ds(0, 8 * C)], vrows, sem).wait()

    def accumulate(wb, wpb, rows, vrows, outf, outv):
        for g in range(C // L):
            s = g * L
            acc = wb[pl.ds(s, L)] * vrows[pl.ds(s, L)]
            for c in range(1, 8):
                acc = acc + wb[pl.ds(c * C + s, L)] * vrows[pl.ds(c * C + s, L)]
            outv[pl.ds(s, L)] = acc

        def pt(j, carry2):
            jv = jnp.full((L,), j, dtype=jnp.int32)
            acc = [None] * (D // L)
            for c2 in range(4):
                ws = plsc.bitcast(
                    plsc.load_gather(wpb, [jv + (c2 * C)]), jnp.bfloat16)
                r = c2 * C + j
                for k in range(D // L):
                    rk = plsc.bitcast(rows[r, pl.ds(k * L, L)], jnp.bfloat16)
                    if c2 == 0:
                        acc[k] = ws * rk
                    else:
                        acc[k] = acc[k] + ws * rk
            for k in range(D // L):
                a, b = plsc.unpack(acc[k], format=plsc.PackFormat.INTERLEAVED)
                outf[j, pl.ds(k * L, L)] = a + b
            return carry2

        lax.fori_loop(0, C, pt, 0, unroll=2)

    def fire_out(i, outf, outv, sem):
        t = base + i * C
        pltpu.async_copy(outf, outf_hbm.at[pl.ds(t, C)], sem)
        pltpu.async_copy(outv, outv_hbm.at[pl.ds(t, C)], sem)

    def wait_out(outf, outv, sem):
        pltpu.make_async_copy(outf, outf_hbm.at[pl.ds(base, C)], sem).wait()
        pltpu.make_async_copy(outv, outv_hbm.at[pl.ds(base, C)], sem).wait()

    idxb0, vidxb0, wb0, wpb0, rows0, vrows0, _, _ = bufs[0]
    compute_idx(0, idxb0, vidxb0, wb0, wpb0)
    fire_gathers(idxb0, vidxb0, rows0, vrows0, semg[0])

    def body2(k, carry):
        for p in (0, 1):
            i = 2 * k + p
            q = 1 - p
            idxb, vidxb, wb, wpb, rows, vrows, outf, outv = bufs[p]
            idxbq, vidxbq, wbq, wpbq, rowsq, vrowsq, _, _ = bufs[q]

            @pl.when(i + 1 < NCHUNK)
            def _prefetch():
                compute_idx(i + 1, idxbq, vidxbq, wbq, wpbq)
                fire_gathers(idxbq, vidxbq, rowsq, vrowsq, semg[q])

            wait_gathers(rows, vrows, semg[p])

            @pl.when(i >= 2)
            def _drain_out():
                wait_out(outf, outv, semo[p])

            accumulate(wb, wpb, rows, vrows, outf, outv)
            fire_out(i, outf, outv, semo[p])
        return carry

    lax.fori_loop(0, NCHUNK // 2, body2, 0)

    for p in (0, 1):
        outf, outv = bufs[p][6], bufs[p][7]
        wait_out(outf, outv, semo[p])


def _parity_bufs():
    return (
        pltpu.VMEM((4, C), jnp.int32),
        pltpu.VMEM((8, C), jnp.int32),
        pltpu.VMEM((8 * C,), jnp.float32),
        pltpu.VMEM((4 * C,), jnp.int32),
        pltpu.VMEM((4 * C, D), jnp.int32),
        pltpu.VMEM((8 * C,), jnp.float32),
        pltpu.VMEM((C, D), jnp.float32),
        pltpu.VMEM((C,), jnp.float32),
    )


_sc_call = pl.kernel(
    _body,
    out_type=(
        jax.ShapeDtypeStruct((B,), jnp.float32),
        jax.ShapeDtypeStruct((B, D), jnp.float32),
    ),
    mesh=plsc.VectorSubcoreMesh(core_axis_name="c", subcore_axis_name="s"),
    compiler_params=pltpu.CompilerParams(needs_layout_passes=False),
    scratch_types=(
        pltpu.VMEM((3 * PT,), jnp.float32),
        *_parity_bufs(),
        *_parity_bufs(),
        pltpu.SemaphoreType.DMA,
        pltpu.SemaphoreType.DMA,
        pltpu.SemaphoreType.DMA,
        pltpu.SemaphoreType.DMA,
    ),
)


@jax.jit
def kernel(x, grid_value_param, grid_feature_param):
    xT = x.T.reshape(-1)
    valt = grid_value_param.reshape(-1)
    ptab = _pack_call(grid_feature_param,
                      grid_feature_param[V - 8:])
    outv, outf = _sc_call(xT, valt, ptab)
    return outv.reshape(B, 1), outf


# speedup vs baseline: 10.1270x; 1.1520x over previous
import jax
import jax.numpy as jnp
from jax import lax
from jax.experimental import pallas as pl
from jax.experimental.pallas import tpu as pltpu
from jax.experimental.pallas import tpu_sc as plsc

N_GRID = 64
N1 = N_GRID + 1
V = N1 * N1 * N1
D = 128
B = 262144
L = 16

NC = 2
NS = 16
NW = NC * NS
PT = B // NW
C = 32
NCHUNK = PT // C

_OFFS = tuple(ox * (N1 * N1) + oy * N1 + oz
              for ox in (0, 1) for oy in (0, 1) for oz in (0, 1))


def _body(xT, valt, feat, outv_hbm, outf_hbm, xv, *bufs_flat):
    semg = bufs_flat[-4:-2]
    semo = bufs_flat[-2:]
    bufs = (bufs_flat[0:6], bufs_flat[6:12])

    wid = lax.axis_index("s") * NC + lax.axis_index("c")
    base = wid * PT

    for d in range(3):
        pltpu.sync_copy(xT.at[pl.ds(d * B + base, PT)],
                        xv.at[pl.ds(d * PT, PT)])

    def compute_idx(i, idxb, wb):
        off = i * C
        for g in range(C // L):
            s = off + g * L
            px = (xv[pl.ds(s, L)] + 1.0) * 32.0
            py = (xv[pl.ds(PT + s, L)] + 1.0) * 32.0
            pz = (xv[pl.ds(2 * PT + s, L)] + 1.0) * 32.0
            ix = px.astype(jnp.int32)
            iy = py.astype(jnp.int32)
            iz = pz.astype(jnp.int32)
            fx = px - ix.astype(jnp.float32)
            fy = py - iy.astype(jnp.float32)
            fz = pz - iz.astype(jnp.float32)
            b0 = ix * (N1 * N1) + iy * N1 + iz
            cidx = 0
            for ox in (0, 1):
                wx = fx if ox else 1.0 - fx
                for oy in (0, 1):
                    wxy = wx * (fy if oy else 1.0 - fy)
                    for oz in (0, 1):
                        w = wxy * (fz if oz else 1.0 - fz)
                        idxb[cidx, pl.ds(g * L, L)] = b0 + _OFFS[cidx]
                        wb[pl.ds(cidx * C + g * L, L)] = w
                        cidx += 1

    def fire_gathers(idxb, rows, vrows, sem):
        for c in range(8):
            pltpu.async_copy(feat.at[idxb.at[c]],
                             rows.at[pl.ds(c * C, C)], sem)
        for c in range(8):
            pltpu.async_copy(valt.at[idxb.at[c]],
                             vrows.at[pl.ds(c * C, C)], sem)

    def wait_gathers(rows, vrows, sem):
        pltpu.make_async_copy(feat.at[pl.ds(0, 8 * C)], rows, sem).wait()
        pltpu.make_async_copy(valt.at[pl.ds(0, 8 * C)], vrows, sem).wait()

    def accumulate(wb, rows, vrows, outf, outv):
        for g in range(C // L):
            s = g * L
            acc = wb[pl.ds(s, L)] * vrows[pl.ds(s, L)]
            for c in range(1, 8):
                acc = acc + wb[pl.ds(c * C + s, L)] * vrows[pl.ds(c * C + s, L)]
            outv[pl.ds(s, L)] = acc

        def pt(j, carry2):
            jv = jnp.full((L,), j, dtype=jnp.int32)
            acc = [None] * (D // L)
            for c in range(8):
                ws = plsc.load_gather(wb, [jv + (c * C)])
                r = c * C + j
                for k in range(D // L):
                    rk = rows[r, pl.ds(k * L, L)]
                    if c == 0:
                        acc[k] = ws * rk
                    else:
                        acc[k] = acc[k] + ws * rk
            for k in range(D // L):
                outf[j, pl.ds(k * L, L)] = acc[k]
            return carry2

        lax.fori_loop(0, C, pt, 0, unroll=2)

    def fire_out(i, outf, outv, sem):
        t = base + i * C
        pltpu.async_copy(outf, outf_hbm.at[pl.ds(t, C)], sem)
        pltpu.async_copy(outv, outv_hbm.at[pl.ds(t, C)], sem)

    def wait_out(outf, outv, sem):
        pltpu.make_async_copy(outf, outf_hbm.at[pl.ds(base, C)], sem).wait()
        pltpu.make_async_copy(outv, outv_hbm.at[pl.ds(base, C)], sem).wait()

    idxb0, wb0, rows0, vrows0, _, _ = bufs[0]
    compute_idx(0, idxb0, wb0)
    fire_gathers(idxb0, rows0, vrows0, semg[0])

    def body2(k, carry):
        for p in (0, 1):
            i = 2 * k + p
            q = 1 - p
            idxb, wb, rows, vrows, outf, outv = bufs[p]
            idxbq, wbq, rowsq, vrowsq, _, _ = bufs[q]

            @pl.when(i + 1 < NCHUNK)
            def _prefetch():
                compute_idx(i + 1, idxbq, wbq)
                fire_gathers(idxbq, rowsq, vrowsq, semg[q])

            wait_gathers(rows, vrows, semg[p])

            @pl.when(i >= 2)
            def _drain_out():
                wait_out(outf, outv, semo[p])

            accumulate(wb, rows, vrows, outf, outv)
            fire_out(i, outf, outv, semo[p])
        return carry

    lax.fori_loop(0, NCHUNK // 2, body2, 0)

    for p in (0, 1):
        outf, outv = bufs[p][4], bufs[p][5]
        wait_out(outf, outv, semo[p])


def _parity_bufs():
    return (
        pltpu.VMEM((8, C), jnp.int32),
        pltpu.VMEM((8 * C,), jnp.float32),
        pltpu.VMEM((8 * C, D), jnp.float32),
        pltpu.VMEM((8 * C,), jnp.float32),
        pltpu.VMEM((C, D), jnp.float32),
        pltpu.VMEM((C,), jnp.float32),
    )


_sc_call = pl.kernel(
    _body,
    out_type=(
        jax.ShapeDtypeStruct((B,), jnp.float32),
        jax.ShapeDtypeStruct((B, D), jnp.float32),
    ),
    mesh=plsc.VectorSubcoreMesh(core_axis_name="c", subcore_axis_name="s"),
    compiler_params=pltpu.CompilerParams(needs_layout_passes=False),
    scratch_types=(
        pltpu.VMEM((3 * PT,), jnp.float32),
        *_parity_bufs(),
        *_parity_bufs(),
        pltpu.SemaphoreType.DMA,
        pltpu.SemaphoreType.DMA,
        pltpu.SemaphoreType.DMA,
        pltpu.SemaphoreType.DMA,
    ),
)


@jax.jit
def kernel(x, grid_value_param, grid_feature_param):
    xT = x.T.reshape(-1)
    valt = grid_value_param.reshape(-1)
    outv, outf = _sc_call(xT, valt, grid_feature_param)
    return outv.reshape(B, 1), outf


# speedup vs baseline: 10.1362x; 1.0009x over previous
import jax
import jax.numpy as jnp
from jax import lax
from jax.experimental import pallas as pl
from jax.experimental.pallas import tpu as pltpu
from jax.experimental.pallas import tpu_sc as plsc

N_GRID = 64
N1 = N_GRID + 1
V = N1 * N1 * N1
D = 128
B = 262144
L = 16

NC = 2
NS = 16
NW = NC * NS
PT = B // NW
C = 32
NCHUNK = PT // C

_OFFS = tuple(ox * (N1 * N1) + oy * N1 + oz
              for ox in (0, 1) for oy in (0, 1) for oz in (0, 1))


def _body(xT, valt, feat, outv_hbm, outf_hbm, xv, *bufs_flat):
    semg = bufs_flat[-4:-2]
    semo = bufs_flat[-2:]
    bufs = (bufs_flat[0:6], bufs_flat[6:12])

    wid = lax.axis_index("s") * NC + lax.axis_index("c")
    base = wid * PT

    for d in range(3):
        pltpu.sync_copy(xT.at[pl.ds(d * B + base, PT)],
                        xv.at[pl.ds(d * PT, PT)])

    def compute_idx(i, idxb, wb):
        off = i * C
        for g in range(C // L):
            s = off + g * L
            px = (xv[pl.ds(s, L)] + 1.0) * 32.0
            py = (xv[pl.ds(PT + s, L)] + 1.0) * 32.0
            pz = (xv[pl.ds(2 * PT + s, L)] + 1.0) * 32.0
            ix = px.astype(jnp.int32)
            iy = py.astype(jnp.int32)
            iz = pz.astype(jnp.int32)
            fx = px - ix.astype(jnp.float32)
            fy = py - iy.astype(jnp.float32)
            fz = pz - iz.astype(jnp.float32)
            b0 = ix * (N1 * N1) + iy * N1 + iz
            cidx = 0
            for ox in (0, 1):
                wx = fx if ox else 1.0 - fx
                for oy in (0, 1):
                    wxy = wx * (fy if oy else 1.0 - fy)
                    for oz in (0, 1):
                        w = wxy * (fz if oz else 1.0 - fz)
                        idxb[pl.ds(cidx * C + g * L, L)] = b0 + _OFFS[cidx]
                        wb[pl.ds(cidx * C + g * L, L)] = w
                        cidx += 1

    def fire_gathers(idxb, rows, vrows, sem):
        for c in range(8):
            pltpu.async_copy(feat.at[idxb.at[pl.ds(c * C, C)]],
                             rows.at[pl.ds(c * C, C)], sem)
        pltpu.async_copy(valt.at[idxb], vrows, sem)

    def wait_gathers(idxb, rows, vrows, sem):
        pltpu.make_async_copy(feat.at[pl.ds(0, 8 * C)], rows, sem).wait()
        pltpu.make_async_copy(valt.at[idxb], vrows, sem).wait()

    def accumulate(wb, rows, vrows, outf, outv):
        for g in range(C // L):
            s = g * L
            acc = wb[pl.ds(s, L)] * vrows[pl.ds(s, L)]
            for c in range(1, 8):
                acc = acc + wb[pl.ds(c * C + s, L)] * vrows[pl.ds(c * C + s, L)]
            outv[pl.ds(s, L)] = acc

        def pt(j, carry2):
            jv = jnp.full((L,), j, dtype=jnp.int32)
            acc = [None] * (D // L)
            for c in range(8):
                ws = plsc.load_gather(wb, [jv + (c * C)])
                r = c * C + j
                for k in range(D // L):
                    rk = rows[r, pl.ds(k * L, L)]
                    if c == 0:
                        acc[k] = ws * rk
                    else:
                        acc[k] = acc[k] + ws * rk
            for k in range(D // L):
                outf[j, pl.ds(k * L, L)] = acc[k]
            return carry2

        lax.fori_loop(0, C, pt, 0, unroll=2)

    def fire_out(i, outf, outv, sem):
        t = base + i * C
        pltpu.async_copy(outf, outf_hbm.at[pl.ds(t, C)], sem)
        pltpu.async_copy(outv, outv_hbm.at[pl.ds(t, C)], sem)

    def wait_out(outf, outv, sem):
        pltpu.make_async_copy(outf, outf_hbm.at[pl.ds(base, C)], sem).wait()
        pltpu.make_async_copy(outv, outv_hbm.at[pl.ds(base, C)], sem).wait()

    idxb0, wb0, rows0, vrows0, _, _ = bufs[0]
    compute_idx(0, idxb0, wb0)
    fire_gathers(idxb0, rows0, vrows0, semg[0])

    def body2(k, carry):
        for p in (0, 1):
            i = 2 * k + p
            q = 1 - p
            idxb, wb, rows, vrows, outf, outv = bufs[p]
            idxbq, wbq, rowsq, vrowsq, _, _ = bufs[q]

            @pl.when(i + 1 < NCHUNK)
            def _prefetch():
                compute_idx(i + 1, idxbq, wbq)
                fire_gathers(idxbq, rowsq, vrowsq, semg[q])

            wait_gathers(idxb, rows, vrows, semg[p])

            @pl.when(i >= 2)
            def _drain_out():
                wait_out(outf, outv, semo[p])

            accumulate(wb, rows, vrows, outf, outv)
            fire_out(i, outf, outv, semo[p])
        return carry

    lax.fori_loop(0, NCHUNK // 2, body2, 0)

    for p in (0, 1):
        outf, outv = bufs[p][4], bufs[p][5]
        wait_out(outf, outv, semo[p])


def _parity_bufs():
    return (
        pltpu.VMEM((8 * C,), jnp.int32),
        pltpu.VMEM((8 * C,), jnp.float32),
        pltpu.VMEM((8 * C, D), jnp.float32),
        pltpu.VMEM((8 * C,), jnp.float32),
        pltpu.VMEM((C, D), jnp.float32),
        pltpu.VMEM((C,), jnp.float32),
    )


_sc_call = pl.kernel(
    _body,
    out_type=(
        jax.ShapeDtypeStruct((B,), jnp.float32),
        jax.ShapeDtypeStruct((B, D), jnp.float32),
    ),
    mesh=plsc.VectorSubcoreMesh(core_axis_name="c", subcore_axis_name="s"),
    compiler_params=pltpu.CompilerParams(needs_layout_passes=False),
    scratch_types=(
        pltpu.VMEM((3 * PT,), jnp.float32),
        *_parity_bufs(),
        *_parity_bufs(),
        pltpu.SemaphoreType.DMA,
        pltpu.SemaphoreType.DMA,
        pltpu.SemaphoreType.DMA,
        pltpu.SemaphoreType.DMA,
    ),
)


@jax.jit
def kernel(x, grid_value_param, grid_feature_param):
    xT = x.T.reshape(-1)
    valt = grid_value_param.reshape(-1)
    outv, outf = _sc_call(xT, valt, grid_feature_param)
    return outv.reshape(B, 1), outf
